# baseline (speedup 1.0000x reference)
import jax
import jax.numpy as jnp
from jax.experimental import pallas as pl
from jax.experimental.pallas import tpu as pltpu

B = 4096
D = 64
KNN = 128
BR = 128
BRG = 256
CT = 1408
ITERS = 6
LAM_ITERS = 100
L2C = 2 * D * D
RC = L2C + 256
NCT = RC // CT
F32 = jnp.float32


def _stats_body(out_ref, tgt_ref, lat_ref, res_ref):
    o = out_ref[...]
    t = tgt_ref[...]
    recon = jnp.sum((o - t) * (o - t)) / (B * D)

    z = lat_ref[...]
    m = jnp.sum(z, axis=0, keepdims=True) / B
    zc = z - m
    cov = jax.lax.dot_general(zc, zc, (((0,), (0,)), ((), ())),
                              preferred_element_type=F32)
    r64 = jax.lax.broadcasted_iota(jnp.int32, (D, D), 0)
    c64 = jax.lax.broadcasted_iota(jnp.int32, (D, D), 1)
    tr = jnp.sum(jnp.where(r64 == c64, cov, 0.0))
    frob2 = jnp.sum(cov * cov)

    v0 = jnp.sum(cov, axis=0, keepdims=True)

    def lam_step(_, v):
        v = jax.lax.dot_general(v, cov, (((1,), (0,)), ((), ())),
                                preferred_element_type=F32)
        return v * jax.lax.rsqrt(jnp.sum(v * v) + 1e-30)

    v = jax.lax.fori_loop(0, LAM_ITERS, lam_step, v0)
    cv = jax.lax.dot_general(v, cov, (((1,), (0,)), ((), ())),
                             preferred_element_type=F32)
    lam = jnp.sum(cv * v) / (jnp.sum(v * v) + 1e-30)

    lane = jax.lax.broadcasted_iota(jnp.int32, (8, 128), 1)
    res_ref[...] = jnp.where(lane == 0, recon,
                    jnp.where(lane == 1, tr,
                     jnp.where(lane == 2, frob2, lam)))


def _w_body(raw_ref, w_ref):
    i = pl.program_id(0)
    rawb = raw_ref[pl.ds(i * BR, BR), :]
    allr = raw_ref[...]
    aa = jnp.sum(rawb * rawb, axis=1, keepdims=True)
    bbc = jnp.sum(allr * allr, axis=1, keepdims=True)
    lhs = jnp.concatenate([allr * (-2.0), bbc, jnp.ones((B, 1), F32)], axis=1)
    rhs = jnp.concatenate([rawb, jnp.ones((BR, 1), F32), aa], axis=1)
    d2 = jax.lax.dot_general(lhs, rhs, (((1,), (1,)), ((), ())),
                             preferred_element_type=F32)
    d2 = jnp.maximum(d2, 0.0)
    rowid = jax.lax.broadcasted_iota(jnp.int32, (B, BR), 0)
    colid = i * BR + jax.lax.broadcasted_iota(jnp.int32, (B, BR), 1)
    self_m = rowid == colid
    d2 = jnp.where(self_m, -1.0, d2)
    hi = jnp.max(d2, axis=0, keepdims=True)
    lo = jnp.zeros((1, BR), F32) - 1.0

    def bis_step(_, lohi):
        lo, hi = lohi
        mid = 0.5 * (lo + hi)
        cnt = jnp.sum(jnp.where(d2 <= mid, 1.0, 0.0), axis=0, keepdims=True)
        pred = cnt < (KNN + 1)
        return jnp.where(pred, mid, lo), jnp.where(pred, hi, mid)

    lo, hi = jax.lax.fori_loop(0, 22, bis_step, (lo, hi))
    w_ref[...] = jnp.where((d2 <= hi) & (~self_m), 1.0, 0.0
                           ).astype(jnp.bfloat16)


def _rhs_body(lat_ref, raw_ref, r_ref):
    zc = lat_ref[...]
    xc = raw_ref[...]
    zx = jnp.concatenate([zc, xc], axis=1)
    nrows = zc.shape[0]
    zhalf = jax.lax.broadcasted_iota(jnp.int32, (nrows, 128), 1) < D
    for i1 in range(0, D, 2):
        f0 = jnp.where(zhalf, zc[:, i1:i1 + 1], xc[:, i1:i1 + 1])
        f1 = jnp.where(zhalf, zc[:, i1 + 1:i1 + 2], xc[:, i1 + 1:i1 + 2])
        r_ref[:, i1 * 128:(i1 + 2) * 128] = (
            jnp.concatenate([zx * f0, zx * f1], axis=1)
            .astype(jnp.bfloat16))
    r_ref[:, L2C:] = jnp.concatenate(
        [zc, xc, jnp.ones((nrows, 128), F32)], axis=1).astype(jnp.bfloat16)


def _gram_body(w_ref, r_ref, g_ref):
    g_ref[...] = jax.lax.dot_general(
        r_ref[...], w_ref[...], (((0,), (0,)), ((), ())),
        preferred_element_type=F32)


def _finish_body(g_ref, tsa_ref):
    i = pl.program_id(0)
    g3 = g_ref[...].reshape(RC // 128, 128, BR)
    l2 = g3[:D]
    szx = g3[D]
    cnt = g3[D + 1][0:1, :]
    inv = 1.0 / cnt
    szT = szx[:D]
    sxT = szx[D:]
    zsub = jax.lax.broadcasted_iota(jnp.int32, (D, 128, BR), 1) < D
    fa = jnp.where(zsub, szT[:, None, :], sxT[:, None, :])
    ct = l2 - fa * (szx[None, :, :] * inv[None, :, :])

    nrm = lambda u: u * jax.lax.rsqrt(
        jnp.sum(u * u, axis=0, keepdims=True) + 1e-30)

    def matvec(v):
        prod = ct * v[None, :, :]
        vz = jnp.sum(prod[:, :D, :], axis=1)
        vx = jnp.sum(prod[:, D:, :], axis=1)
        return vz, vx

    vz0 = nrm(jnp.sum(l2[:, :D, :], axis=1)
              - szT * (jnp.sum(szT, axis=0, keepdims=True) * inv))
    vx0 = nrm(jnp.sum(l2[:, D:, :], axis=1)
              - sxT * (jnp.sum(sxT, axis=0, keepdims=True) * inv))
    v = jnp.concatenate([vz0, vx0], axis=0)

    def pi_step(_, v):
        vz, vx = matvec(v)
        vz, vx = matvec(jnp.concatenate([vz, vx], axis=0))
        return jnp.concatenate([nrm(vz), nrm(vx)], axis=0)

    v = jax.lax.fori_loop(0, ITERS // 2, pi_step, v)
    vz = v[:D]
    vx = v[D:]
    dot = jnp.sum(vz * vx, axis=0, keepdims=True)
    nz = jnp.sum(vz * vz, axis=0, keepdims=True)
    nx = jnp.sum(vx * vx, axis=0, keepdims=True)
    cos2 = dot * dot / (nz * nx + 1e-30)
    part = jnp.sum(2.0 - 2.0 * cos2)

    @pl.when(i == 0)
    def _init_tsa():
        tsa_ref[...] = jnp.zeros((8, 128), F32)

    tsa_ref[...] += part


def kernel(outputs, targets, latent, raw):
    stats = pl.pallas_call(
        _stats_body,
        out_shape=jax.ShapeDtypeStruct((8, 128), F32),
        in_specs=[
            pl.BlockSpec((B, D), lambda: (0, 0)),
            pl.BlockSpec((B, D), lambda: (0, 0)),
            pl.BlockSpec((B, D), lambda: (0, 0)),
        ],
        out_specs=pl.BlockSpec((8, 128), lambda: (0, 0)),
    )(outputs, targets, latent)

    w = pl.pallas_call(
        _w_body,
        grid=(B // BR,),
        out_shape=jax.ShapeDtypeStruct((B, B), jnp.bfloat16),
        in_specs=[pl.BlockSpec((B, D), lambda i: (0, 0))],
        out_specs=pl.BlockSpec((B, BR), lambda i: (0, i)),
        compiler_params=pltpu.CompilerParams(
            dimension_semantics=("arbitrary",),
        ),
    )(raw)

    r = pl.pallas_call(
        _rhs_body,
        grid=(8,),
        out_shape=jax.ShapeDtypeStruct((B, RC), jnp.bfloat16),
        in_specs=[
            pl.BlockSpec((B // 8, D), lambda j: (j, 0)),
            pl.BlockSpec((B // 8, D), lambda j: (j, 0)),
        ],
        out_specs=pl.BlockSpec((B // 8, RC), lambda j: (j, 0)),
        compiler_params=pltpu.CompilerParams(
            dimension_semantics=("arbitrary",),
        ),
    )(latent, raw)

    g = pl.pallas_call(
        _gram_body,
        grid=(NCT, B // BRG),
        out_shape=jax.ShapeDtypeStruct((RC, B), F32),
        in_specs=[
            pl.BlockSpec((B, BRG), lambda c, i: (0, i)),
            pl.BlockSpec((B, CT), lambda c, i: (0, c)),
        ],
        out_specs=pl.BlockSpec((CT, BRG), lambda c, i: (c, i)),
        compiler_params=pltpu.CompilerParams(
            dimension_semantics=("arbitrary", "arbitrary"),
        ),
    )(w, r)

    tsa_acc = pl.pallas_call(
        _finish_body,
        grid=(B // BR,),
        out_shape=jax.ShapeDtypeStruct((8, 128), F32),
        in_specs=[pl.BlockSpec((RC, BR), lambda i: (0, i))],
        out_specs=pl.BlockSpec((8, 128), lambda i: (0, 0)),
        compiler_params=pltpu.CompilerParams(
            dimension_semantics=("arbitrary",),
        ),
    )(g)

    recon = stats[0, 0]
    tr = stats[0, 1]
    frob2 = stats[0, 2]
    lam = stats[0, 3]
    pr = 0.01 * (tr * tr / frob2)
    aniso = 0.01 * (1.0 - lam / tr)
    tsa = 0.1 * (tsa_acc[0, 0] / B)
    return recon + pr + aniso + tsa


# speedup vs baseline: 1.0446x; 1.0446x over previous
import jax
import jax.numpy as jnp
from jax.experimental import pallas as pl
from jax.experimental.pallas import tpu as pltpu

B = 4096
D = 64
KNN = 128
BR = 128
BRG = 256
CT = 1408
ITERS = 6
LAM_ITERS = 100
L2C = 2 * D * D
RC = L2C + 256
NCT = RC // CT
F32 = jnp.float32


def _stats_body(out_ref, tgt_ref, lat_ref, res_ref):
    o = out_ref[...]
    t = tgt_ref[...]
    recon = jnp.sum((o - t) * (o - t)) / (B * D)

    z = lat_ref[...]
    m = jnp.sum(z, axis=0, keepdims=True) / B
    zc = z - m
    cov = jax.lax.dot_general(zc, zc, (((0,), (0,)), ((), ())),
                              preferred_element_type=F32)
    r64 = jax.lax.broadcasted_iota(jnp.int32, (D, D), 0)
    c64 = jax.lax.broadcasted_iota(jnp.int32, (D, D), 1)
    tr = jnp.sum(jnp.where(r64 == c64, cov, 0.0))
    frob2 = jnp.sum(cov * cov)

    v0 = jnp.sum(cov, axis=0, keepdims=True)

    def lam_step(_, v):
        v = jax.lax.dot_general(v, cov, (((1,), (0,)), ((), ())),
                                preferred_element_type=F32)
        return v * jax.lax.rsqrt(jnp.sum(v * v) + 1e-30)

    v = jax.lax.fori_loop(0, LAM_ITERS, lam_step, v0)
    cv = jax.lax.dot_general(v, cov, (((1,), (0,)), ((), ())),
                             preferred_element_type=F32)
    lam = jnp.sum(cv * v) / (jnp.sum(v * v) + 1e-30)

    lane = jax.lax.broadcasted_iota(jnp.int32, (8, 128), 1)
    res_ref[...] = jnp.where(lane == 0, recon,
                    jnp.where(lane == 1, tr,
                     jnp.where(lane == 2, frob2, lam)))


def _w_body(raw_ref, w_ref):
    i = pl.program_id(0)
    rawb = raw_ref[pl.ds(i * BR, BR), :]
    allr = raw_ref[...]
    aa = jnp.sum(rawb * rawb, axis=1, keepdims=True)
    bbc = jnp.sum(allr * allr, axis=1, keepdims=True)
    lhs = jnp.concatenate([rawb * (-2.0), aa, jnp.ones((BR, 1), F32)], axis=1)
    rhs = jnp.concatenate([allr, jnp.ones((B, 1), F32), bbc], axis=1)
    d2 = jax.lax.dot_general(lhs, rhs, (((1,), (1,)), ((), ())),
                             preferred_element_type=F32)
    d2 = jnp.maximum(d2, 0.0)
    di = jax.lax.bitcast_convert_type(d2, jnp.int32)
    rowid = i * BR + jax.lax.broadcasted_iota(jnp.int32, (BR, B), 0)
    colid = jax.lax.broadcasted_iota(jnp.int32, (BR, B), 1)
    di = jnp.where(rowid == colid, jnp.int32(0x7F7FFFFF), di)

    def bit_step(t, T):
        bit = 30 - t
        trial = T | jnp.left_shift(jnp.int32(1), bit)
        cnt = jnp.sum(jnp.where(di <= trial, 1, 0), axis=1, keepdims=True)
        return jnp.where(cnt < KNN, trial, T)

    T = jax.lax.fori_loop(0, 31, bit_step, jnp.zeros((BR, 1), jnp.int32))
    w_ref[...] = jnp.where(di <= T + 1, 1.0, 0.0).astype(jnp.bfloat16)


def _rhs_body(lat_ref, raw_ref, r_ref):
    zc = lat_ref[...]
    xc = raw_ref[...]
    zx = jnp.concatenate([zc, xc], axis=1)
    nrows = zc.shape[0]
    zhalf = jax.lax.broadcasted_iota(jnp.int32, (nrows, 128), 1) < D
    for i1 in range(0, D, 2):
        f0 = jnp.where(zhalf, zc[:, i1:i1 + 1], xc[:, i1:i1 + 1])
        f1 = jnp.where(zhalf, zc[:, i1 + 1:i1 + 2], xc[:, i1 + 1:i1 + 2])
        r_ref[:, i1 * 128:(i1 + 2) * 128] = (
            jnp.concatenate([zx * f0, zx * f1], axis=1)
            .astype(jnp.bfloat16))
    r_ref[:, L2C:] = jnp.concatenate(
        [zc, xc, jnp.ones((nrows, 128), F32)], axis=1).astype(jnp.bfloat16)


def _gram_body(w_ref, r_ref, g_ref):
    p = jax.lax.dot_general(
        w_ref[...], r_ref[...], (((1,), (0,)), ((), ())),
        preferred_element_type=F32)
    g_ref[...] = p.T


def _finish_body(g_ref, tsa_ref):
    i = pl.program_id(0)
    g3 = g_ref[...].reshape(RC // 128, 128, BR)
    l2 = g3[:D]
    szx = g3[D]
    cnt = g3[D + 1][0:1, :]
    inv = 1.0 / cnt
    szT = szx[:D]
    sxT = szx[D:]
    zsub = jax.lax.broadcasted_iota(jnp.int32, (D, 128, BR), 1) < D
    fa = jnp.where(zsub, szT[:, None, :], sxT[:, None, :])
    ct = l2 - fa * (szx[None, :, :] * inv[None, :, :])

    nrm = lambda u: u * jax.lax.rsqrt(
        jnp.sum(u * u, axis=0, keepdims=True) + 1e-30)

    def matvec(v):
        prod = ct * v[None, :, :]
        vz = jnp.sum(prod[:, :D, :], axis=1)
        vx = jnp.sum(prod[:, D:, :], axis=1)
        return vz, vx

    vz0 = nrm(jnp.sum(l2[:, :D, :], axis=1)
              - szT * (jnp.sum(szT, axis=0, keepdims=True) * inv))
    vx0 = nrm(jnp.sum(l2[:, D:, :], axis=1)
              - sxT * (jnp.sum(sxT, axis=0, keepdims=True) * inv))
    v = jnp.concatenate([vz0, vx0], axis=0)

    def pi_step(_, v):
        vz, vx = matvec(v)
        vz, vx = matvec(jnp.concatenate([vz, vx], axis=0))
        return jnp.concatenate([nrm(vz), nrm(vx)], axis=0)

    v = jax.lax.fori_loop(0, ITERS // 2, pi_step, v)
    vz = v[:D]
    vx = v[D:]
    dot = jnp.sum(vz * vx, axis=0, keepdims=True)
    nz = jnp.sum(vz * vz, axis=0, keepdims=True)
    nx = jnp.sum(vx * vx, axis=0, keepdims=True)
    cos2 = dot * dot / (nz * nx + 1e-30)
    part = jnp.sum(2.0 - 2.0 * cos2)

    @pl.when(i == 0)
    def _init_tsa():
        tsa_ref[...] = jnp.zeros((8, 128), F32)

    tsa_ref[...] += part


def kernel(outputs, targets, latent, raw):
    stats = pl.pallas_call(
        _stats_body,
        out_shape=jax.ShapeDtypeStruct((8, 128), F32),
        in_specs=[
            pl.BlockSpec((B, D), lambda: (0, 0)),
            pl.BlockSpec((B, D), lambda: (0, 0)),
            pl.BlockSpec((B, D), lambda: (0, 0)),
        ],
        out_specs=pl.BlockSpec((8, 128), lambda: (0, 0)),
    )(outputs, targets, latent)

    w = pl.pallas_call(
        _w_body,
        grid=(B // BR,),
        out_shape=jax.ShapeDtypeStruct((B, B), jnp.bfloat16),
        in_specs=[pl.BlockSpec((B, D), lambda i: (0, 0))],
        out_specs=pl.BlockSpec((BR, B), lambda i: (i, 0)),
        compiler_params=pltpu.CompilerParams(
            dimension_semantics=("arbitrary",),
        ),
    )(raw)

    r = pl.pallas_call(
        _rhs_body,
        grid=(8,),
        out_shape=jax.ShapeDtypeStruct((B, RC), jnp.bfloat16),
        in_specs=[
            pl.BlockSpec((B // 8, D), lambda j: (j, 0)),
            pl.BlockSpec((B // 8, D), lambda j: (j, 0)),
        ],
        out_specs=pl.BlockSpec((B // 8, RC), lambda j: (j, 0)),
        compiler_params=pltpu.CompilerParams(
            dimension_semantics=("arbitrary",),
        ),
    )(latent, raw)

    g = pl.pallas_call(
        _gram_body,
        grid=(NCT, B // BRG),
        out_shape=jax.ShapeDtypeStruct((RC, B), F32),
        in_specs=[
            pl.BlockSpec((BRG, B), lambda c, i: (i, 0)),
            pl.BlockSpec((B, CT), lambda c, i: (0, c)),
        ],
        out_specs=pl.BlockSpec((CT, BRG), lambda c, i: (c, i)),
        compiler_params=pltpu.CompilerParams(
            dimension_semantics=("arbitrary", "arbitrary"),
        ),
    )(w, r)

    tsa_acc = pl.pallas_call(
        _finish_body,
        grid=(B // BR,),
        out_shape=jax.ShapeDtypeStruct((8, 128), F32),
        in_specs=[pl.BlockSpec((RC, BR), lambda i: (0, i))],
        out_specs=pl.BlockSpec((8, 128), lambda i: (0, 0)),
        compiler_params=pltpu.CompilerParams(
            dimension_semantics=("arbitrary",),
        ),
    )(g)

    recon = stats[0, 0]
    tr = stats[0, 1]
    frob2 = stats[0, 2]
    lam = stats[0, 3]
    pr = 0.01 * (tr * tr / frob2)
    aniso = 0.01 * (1.0 - lam / tr)
    tsa = 0.1 * (tsa_acc[0, 0] / B)
    return recon + pr + aniso + tsa
